# baseline (speedup 1.0000x reference)
import functools

import jax
import jax.numpy as jnp
from jax import lax
from jax.experimental import pallas as pl
from jax.experimental.pallas import tpu as pltpu
from jax.experimental.pallas import tpu_sc as plsc

VOCAB = 1000000
DIM = 64
BATCH = 4096
SEQ = 200

NC = 2
NS = 16
NW = NC * NS
C = BATCH // NW
NCH = SEQ
L = 16
NG = C // L


def _embed_body(idx_hbm, tab_hbm, out_hbm, idx_v, *refs):
    rows = refs[0:2]
    trans = refs[2:4]
    irow = refs[4:6]
    ihalf = refs[6:8]
    gsem = refs[8:10]
    wsem = refs[10:12]
    wid = lax.axis_index("s") * NC + lax.axis_index("c")
    b0 = wid * C

    pltpu.sync_copy(idx_hbm.at[:, pl.ds(b0, C)], idx_v)

    iota = lax.iota(jnp.int32, L)

    def prep_gather(s, b):
        for k in range(NG):
            v = idx_v[s, pl.ds(k * L, L)]
            irow[b][pl.ds(k * L, L)] = v
            ihalf[b][pl.ds(k * L, L)] = lax.shift_right_logical(v, 1)
        pltpu.async_copy(tab_hbm.at[ihalf[b]], rows[b], gsem[b])

    def wait_gather(b):
        pltpu.make_async_copy(tab_hbm.at[ihalf[b]], rows[b], gsem[b]).wait()

    def transpose_chunk(b):
        for k in range(NG):
            off = (irow[b][pl.ds(k * L, L)] & 1) * DIM
            ri = iota + (k * L)
            for d in range(DIM):
                g = plsc.load_gather(rows[b], [ri, off + d])
                trans[b][d, pl.ds(k * L, L)] = g

    def start_write(s, b):
        pltpu.async_copy(trans[b], out_hbm.at[s, :, pl.ds(b0, C)], wsem[b])

    def wait_write(b):
        pltpu.make_async_copy(trans[b], out_hbm.at[0, :, pl.ds(b0, C)], wsem[b]).wait()

    prep_gather(0, 0)
    prep_gather(1, 1)

    def body(k, carry):
        for b in range(2):
            t = 2 * k + b
            wait_gather(b)
            transpose_chunk(b)
            prep_gather(t + 2, b)
            wait_write(b)
            start_write(t, b)
        return carry

    start_write(0, 0)
    start_write(1, 1)

    lax.fori_loop(0, (NCH - 2) // 2, body, 0)

    for b in range(2):
        t = NCH - 2 + b
        wait_gather(b)
        transpose_chunk(b)
        wait_write(b)
        start_write(t, b)
    wait_write(0)
    wait_write(1)


_embed_call = functools.partial(
    pl.kernel,
    mesh=plsc.VectorSubcoreMesh(core_axis_name="c", subcore_axis_name="s"),
    out_type=jax.ShapeDtypeStruct((SEQ, DIM, BATCH), jnp.float32),
    scratch_types=(
        [pltpu.VMEM((NCH, C), jnp.int32)]
        + [pltpu.VMEM((C, 2 * DIM), jnp.float32) for _ in range(2)]
        + [pltpu.VMEM((DIM, C), jnp.float32) for _ in range(2)]
        + [pltpu.VMEM((C,), jnp.int32) for _ in range(4)]
        + [pltpu.SemaphoreType.DMA for _ in range(4)]
    ),
    compiler_params=pltpu.CompilerParams(
        use_tc_tiling_on_sc=True, needs_layout_passes=False),
)(_embed_body)


def kernel(token_ids, table):
    idx_t = token_ids.T.astype(jnp.int32)
    tab_pairs = table.reshape(VOCAB // 2, 2 * DIM)
    out_p = _embed_call(idx_t, tab_pairs)
    return out_p.transpose(2, 0, 1)


# speedup vs baseline: 1.4403x; 1.4403x over previous
import functools

import jax
import jax.numpy as jnp
from jax import lax
from jax.experimental import pallas as pl
from jax.experimental.pallas import tpu as pltpu
from jax.experimental.pallas import tpu_sc as plsc

VOCAB = 1000000
DIM = 64
BATCH = 4096
SEQ = 200

NC = 2
NS = 16
NW = NC * NS
C = BATCH // NW
NCH = SEQ
L = 16
NG = C // L
DB = 8


def _embed_body(idx_hbm, tab_hbm, out_hbm, idx_v, *refs):
    rows = refs[0:2]
    trans = refs[2:4]
    gsem = refs[4:6]
    wsem = refs[6:8]
    wid = lax.axis_index("s") * NC + lax.axis_index("c")
    b0 = wid * C

    pltpu.sync_copy(idx_hbm.at[:, pl.ds(b0, C)], idx_v)

    iota = lax.iota(jnp.int32, L)

    def start_gather(s, b):
        pltpu.async_copy(tab_hbm.at[idx_v.at[s]], rows[b], gsem[b])

    def wait_gather(b):
        pltpu.make_async_copy(tab_hbm.at[idx_v.at[0]], rows[b], gsem[b]).wait()

    def transpose_chunk(b):
        for k in range(NG):
            ri = iota + (k * L)
            for d0 in range(0, DIM, DB):
                g = [plsc.load_gather(rows[b], [ri, jnp.full((L,), d, jnp.int32)])
                     for d in range(d0, d0 + DB)]
                for i in range(DB):
                    trans[b][d0 + i, pl.ds(k * L, L)] = g[i]

    def start_write(s, b):
        pltpu.async_copy(trans[b], out_hbm.at[s, :, pl.ds(b0, C)], wsem[b])

    def wait_write(b):
        pltpu.make_async_copy(trans[b], out_hbm.at[0, :, pl.ds(b0, C)], wsem[b]).wait()

    start_gather(0, 0)
    start_gather(1, 1)

    def body(k, carry):
        for b in range(2):
            t = 2 * k + b
            wait_gather(b)
            transpose_chunk(b)
            start_gather(t + 2, b)
            wait_write(b)
            start_write(t, b)
        return carry

    start_write(0, 0)
    start_write(1, 1)

    lax.fori_loop(0, (NCH - 2) // 2, body, 0)

    for b in range(2):
        t = NCH - 2 + b
        wait_gather(b)
        transpose_chunk(b)
        wait_write(b)
        start_write(t, b)
    wait_write(0)
    wait_write(1)


_embed_call = functools.partial(
    pl.kernel,
    mesh=plsc.VectorSubcoreMesh(core_axis_name="c", subcore_axis_name="s"),
    out_type=jax.ShapeDtypeStruct((SEQ, DIM, BATCH), jnp.float32),
    scratch_types=(
        [pltpu.VMEM((NCH, C), jnp.int32)]
        + [pltpu.VMEM((C, 2 * DIM), jnp.float32) for _ in range(2)]
        + [pltpu.VMEM((DIM, C), jnp.float32) for _ in range(2)]
        + [pltpu.SemaphoreType.DMA for _ in range(4)]
    ),
    compiler_params=pltpu.CompilerParams(
        use_tc_tiling_on_sc=True, needs_layout_passes=False),
)(_embed_body)


def kernel(token_ids, table):
    idx_t = token_ids.T.astype(jnp.int32)
    tab_pad = jnp.pad(table, ((0, 0), (0, DIM)))
    out_p = _embed_call(idx_t, tab_pad)
    return out_p.transpose(2, 0, 1)


# speedup vs baseline: 1.9645x; 1.3640x over previous
import functools

import jax
import jax.numpy as jnp
from jax import lax
from jax.experimental import pallas as pl
from jax.experimental.pallas import tpu as pltpu
from jax.experimental.pallas import tpu_sc as plsc

VOCAB = 1000000
DIM = 64
BATCH = 4096
SEQ = 200

NC = 2
NS = 16
NW = NC * NS
C = BATCH // NW
NCH = SEQ
L = 16
NG = C // L
DB = 8


def _embed_body(idx_hbm, tab_hbm, out_hbm, idx_v, *refs):
    rows = refs[0:2]
    trans = refs[2:4]
    gsem = refs[4:6]
    wsem = refs[6:8]
    wid = lax.axis_index("s") * NC + lax.axis_index("c")
    b0 = wid * C

    pltpu.sync_copy(idx_hbm.at[:, pl.ds(b0, C)], idx_v)

    iota = lax.iota(jnp.int32, L)

    def start_gather(s, b):
        pltpu.async_copy(tab_hbm.at[idx_v.at[s]], rows[b], gsem[b])

    def wait_gather(b):
        pltpu.make_async_copy(tab_hbm.at[idx_v.at[0]], rows[b], gsem[b]).wait()

    zero = jnp.zeros((L,), jnp.int32)
    gb0 = [iota * 128 + ((iota + j) & (L - 1)) for j in range(L)]
    sb0 = [((iota + j) & (L - 1)) * 128 + iota for j in range(L)]

    def transpose_chunk(b):
        def blk(m, carry):
            cg = m // 4
            dg = m - cg * 4
            gv = zero + (cg * (L * 128) + dg * L)
            sv = zero + (dg * (L * 128) + cg * L)
            for j in range(L):
                v = plsc.load_gather(rows[b], [zero, gb0[j] + gv])
                plsc.store_scatter(trans[b], [zero, sb0[j] + sv], v)
            return carry

        lax.fori_loop(0, NG * (DIM // L), blk, 0)

    def start_write(s, b):
        pltpu.async_copy(trans[b], out_hbm.at[s, :, pl.ds(b0, C)], wsem[b])

    def wait_write(b):
        pltpu.make_async_copy(trans[b], out_hbm.at[0, :, pl.ds(b0, C)], wsem[b]).wait()

    start_gather(0, 0)
    start_gather(1, 1)

    def body(k, carry):
        for b in range(2):
            t = 2 * k + b
            wait_gather(b)
            transpose_chunk(b)
            start_gather(t + 2, b)
            wait_write(b)
            start_write(t, b)
        return carry

    start_write(0, 0)
    start_write(1, 1)

    lax.fori_loop(0, (NCH - 2) // 2, body, 0)

    for b in range(2):
        t = NCH - 2 + b
        wait_gather(b)
        transpose_chunk(b)
        wait_write(b)
        start_write(t, b)
    wait_write(0)
    wait_write(1)


_embed_call = functools.partial(
    pl.kernel,
    mesh=plsc.VectorSubcoreMesh(core_axis_name="c", subcore_axis_name="s"),
    out_type=jax.ShapeDtypeStruct((SEQ, DIM, BATCH), jnp.float32),
    scratch_types=(
        [pltpu.VMEM((NCH, C), jnp.int32)]
        + [pltpu.VMEM((C, 2 * DIM), jnp.float32) for _ in range(2)]
        + [pltpu.VMEM((DIM, C), jnp.float32) for _ in range(2)]
        + [pltpu.SemaphoreType.DMA for _ in range(4)]
    ),
    compiler_params=pltpu.CompilerParams(
        use_tc_tiling_on_sc=True, needs_layout_passes=False),
)(_embed_body)


def kernel(token_ids, table):
    idx_t = token_ids.T.astype(jnp.int32)
    tab_pad = jnp.pad(table, ((0, 0), (0, DIM)))
    out_p = _embed_call(idx_t, tab_pad)
    return out_p.transpose(2, 0, 1)


# speedup vs baseline: 1.9855x; 1.0107x over previous
import functools

import jax
import jax.numpy as jnp
from jax import lax
from jax.experimental import pallas as pl
from jax.experimental.pallas import tpu as pltpu
from jax.experimental.pallas import tpu_sc as plsc

VOCAB = 1000000
DIM = 64
BATCH = 4096
SEQ = 200

NC = 2
NS = 16
NW = NC * NS
C = BATCH // NW
NCH = SEQ
L = 16

NWIN = 7813
WPT = 245
VPAD = NWIN * 128

_COMPILER_PARAMS = pltpu.CompilerParams(
    use_tc_tiling_on_sc=True, needs_layout_passes=False)
_MESH = plsc.VectorSubcoreMesh(core_axis_name="c", subcore_axis_name="s")


def _diag_bases(iota):
    gb = [iota * 128 + ((iota + j) & (L - 1)) for j in range(L)]
    sb = [((iota + j) & (L - 1)) * 128 + iota for j in range(L)]
    return gb, sb


def _repack_body(tabt_hbm, out_hbm, wbuf0, wbuf1, tbuf0, tbuf1,
                 g0, g1, w0, w1):
    wbuf = (wbuf0, wbuf1)
    tbuf = (tbuf0, tbuf1)
    gsem = (g0, g1)
    wsem = (w0, w1)
    wid = lax.axis_index("s") * NC + lax.axis_index("c")
    iota = lax.iota(jnp.int32, L)
    zero = jnp.zeros((L,), jnp.int32)
    gb0, sb0 = _diag_bases(iota)

    def col0(i):
        w = jnp.minimum(i * NW + wid, NWIN - 1)
        return w * 128

    def start_read(i, b):
        pltpu.async_copy(tabt_hbm.at[:, pl.ds(col0(i), 128)], wbuf[b], gsem[b])

    def wait_read(b):
        pltpu.make_async_copy(
            tabt_hbm.at[:, pl.ds(0, 128)], wbuf[b], gsem[b]).wait()

    def transpose_win(b):
        def blk(m, carry):
            cg = m // 4
            dg = m - cg * 4
            gv = zero + (dg * (L * 128) + cg * L)
            sv = zero + (cg * (L * 128) + dg * L)
            for j in range(L):
                v = plsc.load_gather(wbuf[b], [zero, sb0[j] + gv])
                plsc.store_scatter(tbuf[b], [zero, gb0[j] + sv], v)
            return carry

        lax.fori_loop(0, 32, blk, 0)

    def start_write(i, b):
        pltpu.async_copy(tbuf[b], out_hbm.at[pl.ds(col0(i), 128)], wsem[b])

    def wait_write(b):
        pltpu.make_async_copy(
            tbuf[b], out_hbm.at[pl.ds(0, 128)], wsem[b]).wait()

    start_read(0, 0)
    start_read(1, 1)
    start_write(0, 0)
    start_write(1, 1)

    def body(k, carry):
        for b in range(2):
            i = 2 * k + b
            wait_read(b)
            transpose_win(b)
            start_read(i + 2, b)
            wait_write(b)
            start_write(i, b)
        return carry

    lax.fori_loop(0, (WPT - 2) // 2, body, 0)
    done = 2 * ((WPT - 2) // 2)
    for i in range(done, WPT):
        b = i % 2
        wait_read(b)
        transpose_win(b)
        if i + 2 < WPT:
            start_read(i + 2, b)
        wait_write(b)
        start_write(i, b)
    wait_write(0)
    wait_write(1)


_repack_call = functools.partial(
    pl.kernel,
    mesh=_MESH,
    out_type=jax.ShapeDtypeStruct((VPAD, 128), jnp.float32),
    scratch_types=(
        [pltpu.VMEM((DIM, 128), jnp.float32) for _ in range(2)]
        + [pltpu.VMEM((128, 128), jnp.float32) for _ in range(2)]
        + [pltpu.SemaphoreType.DMA for _ in range(4)]
    ),
    compiler_params=_COMPILER_PARAMS,
)(_repack_body)


def _embed_body(idx_hbm, tab_hbm, out_hbm, idx_v, *refs):
    rows = refs[0:2]
    trans = refs[2:4]
    gsem = refs[4:6]
    wsem = refs[6:8]
    wid = lax.axis_index("s") * NC + lax.axis_index("c")
    b0 = wid * C

    pltpu.sync_copy(idx_hbm.at[:, pl.ds(b0, C)], idx_v)

    iota = lax.iota(jnp.int32, L)
    zero = jnp.zeros((L,), jnp.int32)
    gb0, sb0 = _diag_bases(iota)

    def start_gather(s, b):
        pltpu.async_copy(tab_hbm.at[idx_v.at[s]], rows[b], gsem[b])

    def wait_gather(b):
        pltpu.make_async_copy(tab_hbm.at[idx_v.at[0]], rows[b], gsem[b]).wait()

    def transpose_chunk(b):
        def blk(m, carry):
            cg = m // 4
            dg = m - cg * 4
            gv = zero + (cg * (L * 128) + dg * L)
            sv = zero + (dg * (L * 128) + cg * L)
            for j in range(L):
                v = plsc.load_gather(rows[b], [zero, gb0[j] + gv])
                plsc.store_scatter(trans[b], [zero, sb0[j] + sv], v)
            return carry

        lax.fori_loop(0, 32, blk, 0)

    def start_write(s, b):
        pltpu.async_copy(trans[b], out_hbm.at[s, :, pl.ds(b0, C)], wsem[b])

    def wait_write(b):
        pltpu.make_async_copy(trans[b], out_hbm.at[0, :, pl.ds(b0, C)], wsem[b]).wait()

    start_gather(0, 0)
    start_gather(1, 1)
    start_write(0, 0)
    start_write(1, 1)

    def body(k, carry):
        for b in range(2):
            t = 2 * k + b
            wait_gather(b)
            transpose_chunk(b)
            start_gather(t + 2, b)
            wait_write(b)
            start_write(t, b)
        return carry

    lax.fori_loop(0, (NCH - 2) // 2, body, 0)

    for b in range(2):
        t = NCH - 2 + b
        wait_gather(b)
        transpose_chunk(b)
        wait_write(b)
        start_write(t, b)
    wait_write(0)
    wait_write(1)


_embed_call = functools.partial(
    pl.kernel,
    mesh=_MESH,
    out_type=jax.ShapeDtypeStruct((SEQ, DIM, BATCH), jnp.float32),
    scratch_types=(
        [pltpu.VMEM((NCH, C), jnp.int32)]
        + [pltpu.VMEM((C, 128), jnp.float32) for _ in range(2)]
        + [pltpu.VMEM((DIM, C), jnp.float32) for _ in range(2)]
        + [pltpu.SemaphoreType.DMA for _ in range(4)]
    ),
    compiler_params=_COMPILER_PARAMS,
)(_embed_body)


def kernel(token_ids, table):
    idx_t = token_ids.T.astype(jnp.int32)
    tab_t = table.T
    tab_rm = _repack_call(tab_t)
    out_p = _embed_call(idx_t, tab_rm)
    return out_p.transpose(2, 0, 1)


# speedup vs baseline: 3.3637x; 1.6941x over previous
import functools

import jax
import jax.numpy as jnp
from jax import lax
from jax.experimental import pallas as pl
from jax.experimental.pallas import tpu as pltpu
from jax.experimental.pallas import tpu_sc as plsc

VOCAB = 1000000
DIM = 64
BATCH = 4096
SEQ = 200

NC = 2
NS = 16
NW = NC * NS
C = BATCH // NW
NCH = SEQ
L = 16

NWIN = 7813
WPT = 245
VPAD = NWIN * 128

_COMPILER_PARAMS = pltpu.CompilerParams(
    use_tc_tiling_on_sc=True, needs_layout_passes=False)
_MESH = plsc.VectorSubcoreMesh(core_axis_name="c", subcore_axis_name="s")


def _diag_bases(iota):
    gb = [iota * 128 + ((iota + j) & (L - 1)) for j in range(L)]
    sb = [((iota + j) & (L - 1)) * 128 + iota for j in range(L)]
    return gb, sb


def _repack_body(tabt_hbm, out_hbm, wbuf0, wbuf1, tbuf0, tbuf1,
                 g0, g1, w0, w1):
    wbuf = (wbuf0, wbuf1)
    tbuf = (tbuf0, tbuf1)
    gsem = (g0, g1)
    wsem = (w0, w1)
    wid = lax.axis_index("s") * NC + lax.axis_index("c")
    iota = lax.iota(jnp.int32, L)
    zero = jnp.zeros((L,), jnp.int32)
    gb0, sb0 = _diag_bases(iota)

    def col0(i):
        w = jnp.minimum(i * NW + wid, NWIN - 1)
        return w * 128

    def start_read(i, b):
        pltpu.async_copy(tabt_hbm.at[:, pl.ds(col0(i), 128)], wbuf[b], gsem[b])

    def wait_read(b):
        pltpu.make_async_copy(
            tabt_hbm.at[:, pl.ds(0, 128)], wbuf[b], gsem[b]).wait()

    def transpose_win(b):
        def blk(m, carry):
            cg = m // 4
            dg = m - cg * 4
            gv = zero + (dg * (L * 128) + cg * L)
            sv = zero + (cg * (L * 128) + dg * L)
            for j0 in range(0, L, 8):
                vs = [plsc.load_gather(wbuf[b], [zero, sb0[j] + gv])
                      for j in range(j0, j0 + 8)]
                for i, j in enumerate(range(j0, j0 + 8)):
                    plsc.store_scatter(tbuf[b], [zero, gb0[j] + sv], vs[i])
            return carry

        lax.fori_loop(0, 32, blk, 0)

    def start_write(i, b):
        pltpu.async_copy(tbuf[b], out_hbm.at[pl.ds(col0(i), 128)], wsem[b])

    def wait_write(b):
        pltpu.make_async_copy(
            tbuf[b], out_hbm.at[pl.ds(0, 128)], wsem[b]).wait()

    start_read(0, 0)
    start_read(1, 1)
    start_write(0, 0)
    start_write(1, 1)

    def body(k, carry):
        for b in range(2):
            i = 2 * k + b
            wait_read(b)
            transpose_win(b)
            start_read(i + 2, b)
            wait_write(b)
            start_write(i, b)
        return carry

    lax.fori_loop(0, (WPT - 2) // 2, body, 0)
    done = 2 * ((WPT - 2) // 2)
    for i in range(done, WPT):
        b = i % 2
        wait_read(b)
        transpose_win(b)
        if i + 2 < WPT:
            start_read(i + 2, b)
        wait_write(b)
        start_write(i, b)
    wait_write(0)
    wait_write(1)


_repack_call = functools.partial(
    pl.kernel,
    mesh=_MESH,
    out_type=jax.ShapeDtypeStruct((VPAD, 128), jnp.float32),
    scratch_types=(
        [pltpu.VMEM((DIM, 128), jnp.float32) for _ in range(2)]
        + [pltpu.VMEM((128, 128), jnp.float32) for _ in range(2)]
        + [pltpu.SemaphoreType.DMA for _ in range(4)]
    ),
    compiler_params=_COMPILER_PARAMS,
)(_repack_body)


def _embed_body(idx_hbm, tab_hbm, out_hbm, idx_v, *refs):
    rows = refs[0:2]
    trans = refs[2:4]
    gsem = refs[4:6]
    wsem = refs[6:8]
    wid = lax.axis_index("s") * NC + lax.axis_index("c")
    b0 = wid * C

    pltpu.sync_copy(idx_hbm.at[:, pl.ds(b0, C)], idx_v)

    iota = lax.iota(jnp.int32, L)
    zero = jnp.zeros((L,), jnp.int32)
    gb0, sb0 = _diag_bases(iota)

    def start_gather(s, b):
        pltpu.async_copy(tab_hbm.at[idx_v.at[s]], rows[b], gsem[b])

    def wait_gather(b):
        pltpu.make_async_copy(tab_hbm.at[idx_v.at[0]], rows[b], gsem[b]).wait()

    def transpose_chunk(b):
        def blk(m, carry):
            cg = m // 4
            dg = m - cg * 4
            gv = zero + (cg * (L * 128) + dg * L)
            sv = zero + (dg * (L * 128) + cg * L)
            for j0 in range(0, L, 8):
                vs = [plsc.load_gather(rows[b], [zero, gb0[j] + gv])
                      for j in range(j0, j0 + 8)]
                for i, j in enumerate(range(j0, j0 + 8)):
                    plsc.store_scatter(trans[b], [zero, sb0[j] + sv], vs[i])
            return carry

        lax.fori_loop(0, 32, blk, 0)

    def start_write(s, b):
        pltpu.async_copy(trans[b], out_hbm.at[s, :, pl.ds(b0, C)], wsem[b])

    def wait_write(b):
        pltpu.make_async_copy(trans[b], out_hbm.at[0, :, pl.ds(b0, C)], wsem[b]).wait()

    start_gather(0, 0)
    start_gather(1, 1)
    start_write(0, 0)
    start_write(1, 1)

    def body(k, carry):
        for b in range(2):
            t = 2 * k + b
            wait_gather(b)
            transpose_chunk(b)
            start_gather(t + 2, b)
            wait_write(b)
            start_write(t, b)
        return carry

    lax.fori_loop(0, (NCH - 2) // 2, body, 0)

    for b in range(2):
        t = NCH - 2 + b
        wait_gather(b)
        transpose_chunk(b)
        wait_write(b)
        start_write(t, b)
    wait_write(0)
    wait_write(1)


_embed_call = functools.partial(
    pl.kernel,
    mesh=_MESH,
    out_type=jax.ShapeDtypeStruct((SEQ, DIM, BATCH), jnp.float32),
    scratch_types=(
        [pltpu.VMEM((NCH, C), jnp.int32)]
        + [pltpu.VMEM((C, 128), jnp.float32) for _ in range(2)]
        + [pltpu.VMEM((DIM, C), jnp.float32) for _ in range(2)]
        + [pltpu.SemaphoreType.DMA for _ in range(4)]
    ),
    compiler_params=_COMPILER_PARAMS,
)(_embed_body)


def kernel(token_ids, table):
    idx_t = token_ids.T.astype(jnp.int32)
    tab_t = table.T
    tab_rm = _repack_call(tab_t)
    out_p = _embed_call(idx_t, tab_rm)
    return out_p.transpose(2, 0, 1)
